# baseline (speedup 1.0000x reference)
import jax
import jax.numpy as jnp
from jax import lax
from jax.experimental import pallas as pl
from jax.experimental.pallas import tpu as pltpu
from jax.experimental.pallas import tpu_sc as plsc

B, L = 4096, 50
N = B * L
D_OUT = 256
NC, NS = 2, 16
NW = NC * NS
PER_W = N // NW
C = 80
NCHUNK = PER_W // C
NPAIR = NCHUNK // 2
G16 = C // 16


def _body(i0, i1, i2, i3, w_id, w_cat, w_br, w_sh, out_hbm,
          jid, jcat, jbr, jsh,
          bid0, bR0, tmpB0, tmpS0, jbp0, jsp0,
          bid1, bR1, tmpB1, tmpS1, jbp1, jsp1,
          gs0, gs1, ws0, ws1):
    wid = lax.axis_index("s") * NC + lax.axis_index("c")
    w_base = wid * PER_W

    pltpu.sync_copy(i0.at[pl.ds(w_base, PER_W)], jid)
    pltpu.sync_copy(i1.at[pl.ds(w_base, PER_W)], jcat)
    pltpu.sync_copy(i2.at[pl.ds(w_base, PER_W)], jbr)
    pltpu.sync_copy(i3.at[pl.ds(w_base, PER_W)], jsh)

    side = [(bid0, bR0, tmpB0, tmpS0, jbp0, jsp0, gs0, ws0),
            (bid1, bR1, tmpB1, tmpS1, jbp1, jsp1, gs1, ws1)]

    def fire_gathers(g, s):
        bid, bR, tmpB, tmpS, jbp, jsp, gs, _ = side[s]
        off = g * C

        def mkidx(gg, c2):
            vb = jbr[pl.ds(off + gg * 16, 16)]
            jbp[pl.ds(gg * 16, 16)] = lax.shift_right_logical(vb, 1)
            vs = jsh[pl.ds(off + gg * 16, 16)]
            jsp[pl.ds(gg * 16, 16)] = lax.shift_right_logical(vs, 2)
            return c2
        lax.fori_loop(0, G16, mkidx, 0)
        pltpu.async_copy(w_id.at[jid.at[pl.ds(off, C)]], bid, gs)
        pltpu.async_copy(w_cat.at[jcat.at[pl.ds(off, C)]], bR, gs)
        pltpu.async_copy(w_br.at[jbp], tmpB, gs)
        pltpu.async_copy(w_sh.at[jsp], tmpS, gs)

    def drain_gathers(g, s):
        bid, bR, tmpB, tmpS, jbp, jsp, gs, _ = side[s]
        off = g * C
        pltpu.make_async_copy(w_id.at[jid.at[pl.ds(off, C)]], bid, gs).wait()
        pltpu.make_async_copy(w_cat.at[jcat.at[pl.ds(off, C)]], bR, gs).wait()
        pltpu.make_async_copy(w_br.at[jbp], tmpB, gs).wait()
        pltpu.make_async_copy(w_sh.at[jsp], tmpS, gs).wait()

    def assemble_fix(g, s):
        bid, bR, tmpB, tmpS, jbp, jsp, _, _ = side[s]
        off0 = g * C

        def group(gg, c2):
            vb = jbr[pl.ds(off0 + gg * 16, 16)]
            vo = (vb & 1) * 64
            vs = jsh[pl.ds(off0 + gg * 16, 16)]
            wo = (vs & 3) * 32
            for l in range(16):
                r = gg * 16 + l
                o = vo[l]
                for j in range(4):
                    bR[r, pl.ds(32 + j * 16, 16)] = tmpB[r, pl.ds(o + j * 16, 16)]
                q = wo[l]
                for j in range(2):
                    bR[r, pl.ds(96 + j * 16, 16)] = tmpS[r, pl.ds(q + j * 16, 16)]
            return c2
        lax.fori_loop(0, G16, group, 0)

        off = g * C
        acc = jnp.zeros((16,), jnp.int32)
        for gg in range(G16):
            iv = jid[pl.ds(off + gg * 16, 16)]
            acc = acc | jnp.where(iv == 0, 1, 0)
        nz = jnp.max(acc)

        @pl.when(nz > 0)
        def _fix():
            def per_group(i, c2):
                iv = jid[pl.ds(off + i * 16, 16)]
                z = iv == 0
                rows = lax.iota(jnp.int32, 16) + i * 16
                zf = jnp.zeros((16,), jnp.float32)
                for col in range(128):
                    cols = jnp.full((16,), col, jnp.int32)
                    plsc.store_scatter(bid, [rows, cols], zf, mask=z)
                return c2
            lax.fori_loop(0, G16, per_group, 0)

    def fire_writes(g, s):
        bid, bR = side[s][0], side[s][1]
        ws = side[s][7]
        base = w_base + g * C
        pltpu.async_copy(bid, out_hbm.at[pl.ds(base, C), pl.ds(0, 128)], ws)
        pltpu.async_copy(bR, out_hbm.at[pl.ds(base, C), pl.ds(128, 128)], ws)

    def drain_writes(g, s):
        bid, bR = side[s][0], side[s][1]
        ws = side[s][7]
        base = w_base + g * C
        pltpu.make_async_copy(
            bid, out_hbm.at[pl.ds(base, C), pl.ds(0, 128)], ws).wait()
        pltpu.make_async_copy(
            bR, out_hbm.at[pl.ds(base, C), pl.ds(128, 128)], ws).wait()

    fire_gathers(0, 0)
    fire_gathers(1, 1)

    def pair(i, carry):
        a = 2 * i
        b = a + 1
        drain_gathers(a, 0)
        assemble_fix(a, 0)
        fire_writes(a, 0)
        drain_gathers(b, 1)
        assemble_fix(b, 1)
        fire_writes(b, 1)
        drain_writes(a, 0)

        @pl.when(i < NPAIR - 1)
        def _n0():
            fire_gathers(a + 2, 0)
        drain_writes(b, 1)

        @pl.when(i < NPAIR - 1)
        def _n1():
            fire_gathers(b + 2, 1)
        return carry

    lax.fori_loop(0, NPAIR, pair, 0)


def kernel(attr_id, attr_category, attr_brand, attr_shop,
           W_id, W_category, W_brand, W_shop):
    ii = attr_id.astype(jnp.int32).reshape(N)
    ic = attr_category.astype(jnp.int32).reshape(N)
    ib = attr_brand.astype(jnp.int32).reshape(N)
    ish = attr_shop.astype(jnp.int32).reshape(N)
    w_cat = jnp.pad(W_category, ((0, 0), (0, 96)))
    w_br = W_brand.reshape(50000, 128)
    w_sh = W_shop.reshape(2500, 128)
    k = pl.kernel(
        _body,
        out_type=jax.ShapeDtypeStruct((N, D_OUT), jnp.float32),
        mesh=plsc.VectorSubcoreMesh(core_axis_name="c", subcore_axis_name="s"),
        compiler_params=pltpu.CompilerParams(needs_layout_passes=False),
        scratch_types=[
            pltpu.VMEM((PER_W,), jnp.int32),
            pltpu.VMEM((PER_W,), jnp.int32),
            pltpu.VMEM((PER_W,), jnp.int32),
            pltpu.VMEM((PER_W,), jnp.int32),
            pltpu.VMEM((C, 128), jnp.float32),
            pltpu.VMEM((C, 128), jnp.float32),
            pltpu.VMEM((C, 128), jnp.float32),
            pltpu.VMEM((C, 128), jnp.float32),
            pltpu.VMEM((C,), jnp.int32),
            pltpu.VMEM((C,), jnp.int32),
            pltpu.VMEM((C, 128), jnp.float32),
            pltpu.VMEM((C, 128), jnp.float32),
            pltpu.VMEM((C, 128), jnp.float32),
            pltpu.VMEM((C, 128), jnp.float32),
            pltpu.VMEM((C,), jnp.int32),
            pltpu.VMEM((C,), jnp.int32),
            pltpu.SemaphoreType.DMA,
            pltpu.SemaphoreType.DMA,
            pltpu.SemaphoreType.DMA,
            pltpu.SemaphoreType.DMA,
        ],
    )
    out = k(ii, ic, ib, ish, W_id, w_cat, w_br, w_sh)
    return out.reshape(B, L, D_OUT)


# speedup vs baseline: 1.0386x; 1.0386x over previous
import jax
import jax.numpy as jnp
from jax import lax
from jax.experimental import pallas as pl
from jax.experimental.pallas import tpu as pltpu
from jax.experimental.pallas import tpu_sc as plsc

B, L = 4096, 50
N = B * L
D_OUT = 256
NC, NS = 2, 16
NW = NC * NS
PER_W = N // NW
C = 80
NCHUNK = PER_W // C
NPAIR = NCHUNK // 2
G16 = C // 16


def _body(i0, i1, i2, i3, w_id, w_cat, w_br, w_sh, out_hbm,
          jid, jcat, jbr, jsh,
          bid0, bR0, tmpB0, tmpS0, jbp0, jsp0,
          bid1, bR1, tmpB1, tmpS1, jbp1, jsp1,
          gs0, gs1, ws0, ws1):
    wid = lax.axis_index("s") * NC + lax.axis_index("c")
    w_base = wid * PER_W

    pltpu.sync_copy(i0.at[pl.ds(w_base, PER_W)], jid)
    pltpu.sync_copy(i1.at[pl.ds(w_base, PER_W)], jcat)
    pltpu.sync_copy(i2.at[pl.ds(w_base, PER_W)], jbr)
    pltpu.sync_copy(i3.at[pl.ds(w_base, PER_W)], jsh)

    side = [(bid0, bR0, tmpB0, tmpS0, jbp0, jsp0, gs0, ws0),
            (bid1, bR1, tmpB1, tmpS1, jbp1, jsp1, gs1, ws1)]

    def fire_gathers(g, s):
        bid, bR, tmpB, tmpS, jbp, jsp, gs, _ = side[s]
        off = g * C
        pltpu.async_copy(w_id.at[jid.at[pl.ds(off, C)]], bid, gs)
        pltpu.async_copy(w_cat.at[jcat.at[pl.ds(off, C)]], bR, gs)
        pltpu.async_copy(w_br.at[jbr.at[pl.ds(off, C)]], tmpB, gs)
        pltpu.async_copy(w_sh.at[jsh.at[pl.ds(off, C)]], tmpS, gs)

    def drain_gathers(g, s):
        bid, bR, tmpB, tmpS, jbp, jsp, gs, _ = side[s]
        off = g * C
        pltpu.make_async_copy(w_id.at[jid.at[pl.ds(off, C)]], bid, gs).wait()
        pltpu.make_async_copy(w_cat.at[jcat.at[pl.ds(off, C)]], bR, gs).wait()
        pltpu.make_async_copy(w_br.at[jbr.at[pl.ds(off, C)]], tmpB, gs).wait()
        pltpu.make_async_copy(w_sh.at[jsh.at[pl.ds(off, C)]], tmpS, gs).wait()

    def assemble_fix(g, s):
        bid, bR, tmpB, tmpS, jbp, jsp, _, _ = side[s]
        off = g * C

        def row(r, c2):
            for j in range(4):
                bR[r, pl.ds(32 + j * 16, 16)] = tmpB[r, pl.ds(j * 16, 16)]
            for j in range(2):
                bR[r, pl.ds(96 + j * 16, 16)] = tmpS[r, pl.ds(j * 16, 16)]
            return c2
        lax.fori_loop(0, C, row, 0)

        off = g * C
        acc = jnp.zeros((16,), jnp.int32)
        for gg in range(G16):
            iv = jid[pl.ds(off + gg * 16, 16)]
            acc = acc | jnp.where(iv == 0, 1, 0)
        nz = jnp.max(acc)

        @pl.when(nz > 0)
        def _fix():
            def per_group(i, c2):
                iv = jid[pl.ds(off + i * 16, 16)]
                z = iv == 0
                rows = lax.iota(jnp.int32, 16) + i * 16
                zf = jnp.zeros((16,), jnp.float32)
                for col in range(128):
                    cols = jnp.full((16,), col, jnp.int32)
                    plsc.store_scatter(bid, [rows, cols], zf, mask=z)
                return c2
            lax.fori_loop(0, G16, per_group, 0)

    def fire_writes(g, s):
        bid, bR = side[s][0], side[s][1]
        ws = side[s][7]
        base = w_base + g * C
        pltpu.async_copy(bid, out_hbm.at[pl.ds(base, C), pl.ds(0, 128)], ws)
        pltpu.async_copy(bR, out_hbm.at[pl.ds(base, C), pl.ds(128, 128)], ws)

    def drain_writes(g, s):
        bid, bR = side[s][0], side[s][1]
        ws = side[s][7]
        base = w_base + g * C
        pltpu.make_async_copy(
            bid, out_hbm.at[pl.ds(base, C), pl.ds(0, 128)], ws).wait()
        pltpu.make_async_copy(
            bR, out_hbm.at[pl.ds(base, C), pl.ds(128, 128)], ws).wait()

    fire_gathers(0, 0)
    fire_gathers(1, 1)

    def pair(i, carry):
        a = 2 * i
        b = a + 1
        drain_gathers(a, 0)
        assemble_fix(a, 0)
        fire_writes(a, 0)
        drain_gathers(b, 1)
        assemble_fix(b, 1)
        fire_writes(b, 1)
        drain_writes(a, 0)

        @pl.when(i < NPAIR - 1)
        def _n0():
            fire_gathers(a + 2, 0)
        drain_writes(b, 1)

        @pl.when(i < NPAIR - 1)
        def _n1():
            fire_gathers(b + 2, 1)
        return carry

    lax.fori_loop(0, NPAIR, pair, 0)


def _pad_body(src, dst, nb0, pb0, nb1, pb1, rs0, rs1, ws0, ws1, V, D, K):
    wid = lax.axis_index("s") * 2 + lax.axis_index("c")
    nblk = V // K
    nvec = D // 16
    sides = [(nb0, pb0, rs0, ws0), (nb1, pb1, rs1, ws1)]

    nmine = (nblk - wid + NW - 1) // NW

    def fire_read(t, s):
        nb = sides[s][0]
        rs = sides[s][2]
        b = wid + t * NW
        pltpu.async_copy(src.at[pl.ds(b * K, K)], nb, rs)

    def drain_read(t, s):
        nb = sides[s][0]
        rs = sides[s][2]
        b = wid + t * NW
        pltpu.make_async_copy(src.at[pl.ds(b * K, K)], nb, rs).wait()

    def fire_write(t, s):
        pb = sides[s][1]
        ws = sides[s][3]
        b = wid + t * NW
        pltpu.async_copy(pb, dst.at[pl.ds(b * K, K)], ws)

    def drain_write(t, s):
        pb = sides[s][1]
        ws = sides[s][3]
        b = wid + t * NW
        pltpu.make_async_copy(pb, dst.at[pl.ds(b * K, K)], ws).wait()

    zv = jnp.zeros((16,), jnp.float32)

    def assemble(s):
        nb, pb = sides[s][0], sides[s][1]

        def row(r, c2):
            for j in range(nvec):
                pb[r, pl.ds(j * 16, 16)] = nb[r, pl.ds(j * 16, 16)]
            for j in range(nvec, 8):
                pb[r, pl.ds(j * 16, 16)] = zv
            return c2
        lax.fori_loop(0, K, row, 0)

    @pl.when(nmine > 0)
    def _prologue():
        fire_read(0, 0)

    @pl.when(nmine > 1)
    def _prologue2():
        fire_read(1, 1)

    def half(t, s):
        @pl.when(t < nmine)
        def _do():
            drain_read(t, s)
            assemble(s)

            @pl.when(t >= 2)
            def _dw():
                drain_write(t - 2, s)
            fire_write(t, s)

            @pl.when(t + 2 < nmine)
            def _nx():
                fire_read(t + 2, s)

    def step(p, c):
        half(2 * p, 0)
        half(2 * p + 1, 1)
        return c

    nloop = (nblk + NW - 1) // NW
    lax.fori_loop(0, (nloop + 1) // 2, step, 0)

    for s_ in (0, 1):
        t_s = 2 * ((nmine - 1 - s_) // 2) + s_

        @pl.when(nmine > s_)
        def _ep(t_s=t_s, s_=s_):
            drain_write(t_s, s_)


def _pad_one(src, V, D, K):
    import functools
    body = functools.partial(_pad_body, V=V, D=D, K=K)
    k = pl.kernel(
        body,
        out_type=jax.ShapeDtypeStruct((V, 128), jnp.float32),
        mesh=plsc.VectorSubcoreMesh(core_axis_name="c", subcore_axis_name="s"),
        compiler_params=pltpu.CompilerParams(needs_layout_passes=False),
        scratch_types=[
            pltpu.VMEM((K, D), jnp.float32),
            pltpu.VMEM((K, 128), jnp.float32),
            pltpu.VMEM((K, D), jnp.float32),
            pltpu.VMEM((K, 128), jnp.float32),
            pltpu.SemaphoreType.DMA,
            pltpu.SemaphoreType.DMA,
            pltpu.SemaphoreType.DMA,
            pltpu.SemaphoreType.DMA,
        ],
    )
    return k(src)


def kernel(attr_id, attr_category, attr_brand, attr_shop,
           W_id, W_category, W_brand, W_shop):
    ii = attr_id.astype(jnp.int32).reshape(N)
    ic = attr_category.astype(jnp.int32).reshape(N)
    ib = attr_brand.astype(jnp.int32).reshape(N)
    ish = attr_shop.astype(jnp.int32).reshape(N)
    w_cat = _pad_one(W_category, 1000, 32, 40)
    w_br = _pad_one(W_brand, 100000, 64, 80)
    w_sh = _pad_one(W_shop, 10000, 32, 80)
    k = pl.kernel(
        _body,
        out_type=jax.ShapeDtypeStruct((N, D_OUT), jnp.float32),
        mesh=plsc.VectorSubcoreMesh(core_axis_name="c", subcore_axis_name="s"),
        compiler_params=pltpu.CompilerParams(needs_layout_passes=False),
        scratch_types=[
            pltpu.VMEM((PER_W,), jnp.int32),
            pltpu.VMEM((PER_W,), jnp.int32),
            pltpu.VMEM((PER_W,), jnp.int32),
            pltpu.VMEM((PER_W,), jnp.int32),
            pltpu.VMEM((C, 128), jnp.float32),
            pltpu.VMEM((C, 128), jnp.float32),
            pltpu.VMEM((C, 128), jnp.float32),
            pltpu.VMEM((C, 128), jnp.float32),
            pltpu.VMEM((C,), jnp.int32),
            pltpu.VMEM((C,), jnp.int32),
            pltpu.VMEM((C, 128), jnp.float32),
            pltpu.VMEM((C, 128), jnp.float32),
            pltpu.VMEM((C, 128), jnp.float32),
            pltpu.VMEM((C, 128), jnp.float32),
            pltpu.VMEM((C,), jnp.int32),
            pltpu.VMEM((C,), jnp.int32),
            pltpu.SemaphoreType.DMA,
            pltpu.SemaphoreType.DMA,
            pltpu.SemaphoreType.DMA,
            pltpu.SemaphoreType.DMA,
        ],
    )
    out = k(ii, ic, ib, ish, W_id, w_cat, w_br, w_sh)
    return out.reshape(B, L, D_OUT)


# speedup vs baseline: 1.0996x; 1.0587x over previous
import jax
import jax.numpy as jnp
from jax import lax
from jax.experimental import pallas as pl
from jax.experimental.pallas import tpu as pltpu
from jax.experimental.pallas import tpu_sc as plsc

B, L = 4096, 50
N = B * L
D_OUT = 256
NC, NS = 2, 16
NW = NC * NS
PER_W = N // NW
C = 80
NCHUNK = PER_W // C
NPAIR = NCHUNK // 2
G16 = C // 16


def _body(i0, i1, i2, i3, w_id, w_cat, w_br, w_sh, out_hbm,
          jid, jcat, jbr, jsh,
          bid0, bR0, tmpB0, tmpS0, jbp0, jsp0,
          bid1, bR1, tmpB1, tmpS1, jbp1, jsp1,
          gs0, gs1, ws0, ws1):
    wid = lax.axis_index("s") * NC + lax.axis_index("c")
    w_base = wid * PER_W

    pltpu.sync_copy(i0.at[pl.ds(w_base, PER_W)], jid)
    pltpu.sync_copy(i1.at[pl.ds(w_base, PER_W)], jcat)
    pltpu.sync_copy(i2.at[pl.ds(w_base, PER_W)], jbr)
    pltpu.sync_copy(i3.at[pl.ds(w_base, PER_W)], jsh)

    side = [(bid0, bR0, tmpB0, tmpS0, jbp0, jsp0, gs0, ws0),
            (bid1, bR1, tmpB1, tmpS1, jbp1, jsp1, gs1, ws1)]

    def fire_gathers(g, s):
        bid, bR, tmpB, tmpS, jbp, jsp, gs, _ = side[s]
        off = g * C
        pltpu.async_copy(w_id.at[jid.at[pl.ds(off, C)]], bid, gs)
        pltpu.async_copy(w_cat.at[jcat.at[pl.ds(off, C)]], bR, gs)
        pltpu.async_copy(w_br.at[jbr.at[pl.ds(off, C)]], tmpB, gs)
        pltpu.async_copy(w_sh.at[jsh.at[pl.ds(off, C)]], tmpS, gs)

    def drain_gathers(g, s):
        bid, bR, tmpB, tmpS, jbp, jsp, gs, _ = side[s]
        off = g * C
        pltpu.make_async_copy(w_id.at[jid.at[pl.ds(off, C)]], bid, gs).wait()
        pltpu.make_async_copy(w_cat.at[jcat.at[pl.ds(off, C)]], bR, gs).wait()
        pltpu.make_async_copy(w_br.at[jbr.at[pl.ds(off, C)]], tmpB, gs).wait()
        pltpu.make_async_copy(w_sh.at[jsh.at[pl.ds(off, C)]], tmpS, gs).wait()

    def assemble_fix(g, s):
        bid, bR, tmpB, tmpS, jbp, jsp, _, _ = side[s]
        off = g * C

        def row(r, c2):
            for j in range(4):
                bR[r, pl.ds(32 + j * 16, 16)] = tmpB[r, pl.ds(j * 16, 16)]
            for j in range(2):
                bR[r, pl.ds(96 + j * 16, 16)] = tmpS[r, pl.ds(j * 16, 16)]
            return c2
        lax.fori_loop(0, C, row, 0)

        off = g * C
        acc = jnp.zeros((16,), jnp.int32)
        for gg in range(G16):
            iv = jid[pl.ds(off + gg * 16, 16)]
            acc = acc | jnp.where(iv == 0, 1, 0)
        nz = jnp.max(acc)

        @pl.when(nz > 0)
        def _fix():
            def per_group(i, c2):
                iv = jid[pl.ds(off + i * 16, 16)]
                z = iv == 0
                rows = lax.iota(jnp.int32, 16) + i * 16
                zf = jnp.zeros((16,), jnp.float32)
                for col in range(128):
                    cols = jnp.full((16,), col, jnp.int32)
                    plsc.store_scatter(bid, [rows, cols], zf, mask=z)
                return c2
            lax.fori_loop(0, G16, per_group, 0)

    def fire_writes(g, s):
        bid, bR = side[s][0], side[s][1]
        ws = side[s][7]
        base = w_base + g * C
        pltpu.async_copy(bid, out_hbm.at[pl.ds(base, C), pl.ds(0, 128)], ws)
        pltpu.async_copy(bR, out_hbm.at[pl.ds(base, C), pl.ds(128, 128)], ws)

    def drain_writes(g, s):
        bid, bR = side[s][0], side[s][1]
        ws = side[s][7]
        base = w_base + g * C
        pltpu.make_async_copy(
            bid, out_hbm.at[pl.ds(base, C), pl.ds(0, 128)], ws).wait()
        pltpu.make_async_copy(
            bR, out_hbm.at[pl.ds(base, C), pl.ds(128, 128)], ws).wait()

    fire_gathers(0, 0)
    fire_gathers(1, 1)

    def pair(i, carry):
        a = 2 * i
        b = a + 1
        drain_gathers(a, 0)
        assemble_fix(a, 0)
        fire_writes(a, 0)
        drain_gathers(b, 1)
        assemble_fix(b, 1)
        fire_writes(b, 1)
        drain_writes(a, 0)

        @pl.when(i < NPAIR - 1)
        def _n0():
            fire_gathers(a + 2, 0)
        drain_writes(b, 1)

        @pl.when(i < NPAIR - 1)
        def _n1():
            fire_gathers(b + 2, 1)
        return carry

    lax.fori_loop(0, NPAIR, pair, 0)


def kernel(attr_id, attr_category, attr_brand, attr_shop,
           W_id, W_category, W_brand, W_shop):
    ii = attr_id.astype(jnp.int32).reshape(N)
    ic = attr_category.astype(jnp.int32).reshape(N)
    ib = attr_brand.astype(jnp.int32).reshape(N)
    ish = attr_shop.astype(jnp.int32).reshape(N)
    w_cat = jnp.pad(W_category, ((0, 0), (0, 96)))
    w_br = jnp.pad(W_brand, ((0, 0), (0, 64)))
    w_sh = jnp.pad(W_shop, ((0, 0), (0, 96)))
    k = pl.kernel(
        _body,
        out_type=jax.ShapeDtypeStruct((N, D_OUT), jnp.float32),
        mesh=plsc.VectorSubcoreMesh(core_axis_name="c", subcore_axis_name="s"),
        compiler_params=pltpu.CompilerParams(needs_layout_passes=False),
        scratch_types=[
            pltpu.VMEM((PER_W,), jnp.int32),
            pltpu.VMEM((PER_W,), jnp.int32),
            pltpu.VMEM((PER_W,), jnp.int32),
            pltpu.VMEM((PER_W,), jnp.int32),
            pltpu.VMEM((C, 128), jnp.float32),
            pltpu.VMEM((C, 128), jnp.float32),
            pltpu.VMEM((C, 128), jnp.float32),
            pltpu.VMEM((C, 128), jnp.float32),
            pltpu.VMEM((C,), jnp.int32),
            pltpu.VMEM((C,), jnp.int32),
            pltpu.VMEM((C, 128), jnp.float32),
            pltpu.VMEM((C, 128), jnp.float32),
            pltpu.VMEM((C, 128), jnp.float32),
            pltpu.VMEM((C, 128), jnp.float32),
            pltpu.VMEM((C,), jnp.int32),
            pltpu.VMEM((C,), jnp.int32),
            pltpu.SemaphoreType.DMA,
            pltpu.SemaphoreType.DMA,
            pltpu.SemaphoreType.DMA,
            pltpu.SemaphoreType.DMA,
        ],
    )
    out = k(ii, ic, ib, ish, W_id, w_cat, w_br, w_sh)
    return out.reshape(B, L, D_OUT)
